# Initial kernel scaffold; baseline (speedup 1.0000x reference)
import jax
import jax.numpy as jnp
from jax.experimental import pallas as pl


def kernel(x, edge_index, edge_type, W1, root1, b1, g1, beta1, W2, root2, b2, g2, beta2):
    raise NotImplementedError("write your pallas kernel here")


# speedup vs baseline: 11.3726x; 11.3726x over previous
import jax
import jax.numpy as jnp
from jax.experimental import pallas as pl

_N = 2048
_R = 3
_E_TILE = 512
_ROW_TILE = 256


def _agg_kernel(src_ref, dst_ref, et_ref, keep_ref, x_ref, agg_ref, cnt_ref):
    t = pl.program_id(0)

    @pl.when(t == 0)
    def _init():
        agg_ref[...] = jnp.zeros_like(agg_ref)
        cnt_ref[...] = jnp.zeros_like(cnt_ref)

    srow = src_ref[pl.ds(t, 1), :]
    drow = dst_ref[pl.ds(t, 1), :]
    erow = et_ref[pl.ds(t, 1), :]
    krow = keep_ref[pl.ds(t, 1), :]

    ids = jax.lax.broadcasted_iota(jnp.int32, (_N, srow.shape[1]), 0)
    src_ohT = (ids == srow).astype(jnp.float32)
    dst_ohT = (ids == drow).astype(jnp.float32)
    msg = jax.lax.dot_general(src_ohT, x_ref[...], (((0,), (0,)), ((), ())),
                              preferred_element_type=jnp.float32)
    for r in range(_R):
        m = jnp.where(erow == r, krow, 0.0)
        doh = dst_ohT * m
        contrib = jnp.dot(doh, msg, preferred_element_type=jnp.float32)
        agg_ref[r] += contrib
        cnt_ref[:, r:r + 1] += jnp.sum(doh, axis=1, keepdims=True)


def _combine_kernel(xin_ref, agg_ref, cnt_ref, w_ref, root_ref, b_ref,
                    g_ref, beta_ref, out_ref):
    z = jnp.dot(xin_ref[...], root_ref[...],
                preferred_element_type=jnp.float32) + b_ref[...]
    for r in range(_R):
        c = jnp.maximum(cnt_ref[:, r:r + 1], 1.0)
        mean_r = agg_ref[r] / c
        z += jnp.dot(mean_r, w_ref[r], preferred_element_type=jnp.float32)
    z = jnp.maximum(z, 0.0)
    mu = jnp.mean(z, axis=1, keepdims=True)
    zc = z - mu
    var = jnp.mean(zc * zc, axis=1, keepdims=True)
    out_ref[...] = zc / jnp.sqrt(var + 1e-5) * g_ref[...] + beta_ref[...]


def _push_kernel(a_ref, h_ref, out_ref):
    t = pl.program_id(0)
    at = a_ref[...]
    h = h_ref[...]
    ht = h_ref[pl.ds(t * _ROW_TILE, _ROW_TILE), :]
    d2 = jnp.sum(at, axis=1, keepdims=True)
    d2 = jnp.where(d2 == 0.0, 1.0, d2)
    prop = jnp.dot(at, h, preferred_element_type=jnp.float32) / d2
    out_ref[...] = 0.9 * ht + 0.1 * prop


def _topk_kernel(x_ref, h_ref, out_ref):
    t = pl.program_id(0)
    k = _N // 2
    xt = x_ref[pl.ds(t * _ROW_TILE, _ROW_TILE), :]
    adj = jax.lax.dot_general(xt, x_ref[...], (((1,), (1,)), ((), ())),
                              preferred_element_type=jnp.float32)
    col = jax.lax.broadcasted_iota(jnp.int32, (_ROW_TILE, _N), 1)
    row = jax.lax.broadcasted_iota(jnp.int32, (_ROW_TILE, _N), 0) + t * _ROW_TILE
    adj = jnp.where(row == col, 0.0, adj)
    deg = jnp.sum(adj, axis=1, keepdims=True)
    deg = jnp.where(deg == 0.0, 1.0, deg)
    p = adj / deg
    u = jax.lax.bitcast_convert_type(p, jnp.int32)
    key = u ^ ((u >> 31) & jnp.int32(0x7FFFFFFF))
    nonneg = key >= 0
    sign_cnt = jnp.sum(nonneg.astype(jnp.int32), axis=1, keepdims=True)
    use_neg = sign_cnt < k
    kk = jnp.where(use_neg, k - sign_cnt, k)
    active = jnp.logical_xor(nonneg, use_neg)
    low = key & jnp.int32(0x7FFFFFFF)
    thr = jnp.zeros((_ROW_TILE, 1), jnp.int32)
    for b in range(30, -1, -1):
        cand = thr | jnp.int32(1 << b)
        ge = jnp.logical_and(active, low >= cand)
        c = jnp.sum(ge.astype(jnp.int32), axis=1, keepdims=True)
        thr = jnp.where(c >= kk, cand, thr)
    tkey = jnp.where(use_neg, thr | jnp.int32(-2147483648), thr)
    sp = jnp.where(key >= tkey, p, 0.0)
    out_ref[...] = jnp.dot(sp, h_ref[...], preferred_element_type=jnp.float32)


def _rgcn_agg(src2, dst2, et2, keep2, xin):
    eb = src2.shape[0]
    n, d = xin.shape
    idx_spec = pl.BlockSpec(src2.shape, lambda t: (0, 0))
    return pl.pallas_call(
        _agg_kernel,
        grid=(eb,),
        in_specs=[idx_spec, idx_spec, idx_spec, idx_spec,
                  pl.BlockSpec((n, d), lambda t: (0, 0))],
        out_specs=[pl.BlockSpec((_R, n, d), lambda t: (0, 0, 0)),
                   pl.BlockSpec((n, 8), lambda t: (0, 0))],
        out_shape=[jax.ShapeDtypeStruct((_R, n, d), jnp.float32),
                   jax.ShapeDtypeStruct((n, 8), jnp.float32)],
    )(src2, dst2, et2, keep2, xin)


def _rgcn_combine(xin, agg, cnt, w, root, b, g, beta):
    n, d = xin.shape
    h = w.shape[2]
    vec = pl.BlockSpec((1, h), lambda t: (0, 0))
    return pl.pallas_call(
        _combine_kernel,
        grid=(n // _ROW_TILE,),
        in_specs=[pl.BlockSpec((_ROW_TILE, d), lambda t: (t, 0)),
                  pl.BlockSpec((_R, _ROW_TILE, d), lambda t: (0, t, 0)),
                  pl.BlockSpec((_ROW_TILE, 8), lambda t: (t, 0)),
                  pl.BlockSpec((_R, d, h), lambda t: (0, 0, 0)),
                  pl.BlockSpec((d, h), lambda t: (0, 0)),
                  vec, vec, vec],
        out_specs=pl.BlockSpec((_ROW_TILE, h), lambda t: (t, 0)),
        out_shape=jax.ShapeDtypeStruct((n, h), jnp.float32),
    )(xin, agg, cnt, w, root, b.reshape(1, h), g.reshape(1, h),
      beta.reshape(1, h))


def _push(a, h):
    n, hd = h.shape
    return pl.pallas_call(
        _push_kernel,
        grid=(n // _ROW_TILE,),
        in_specs=[pl.BlockSpec((_ROW_TILE, n), lambda t: (t, 0)),
                  pl.BlockSpec((n, hd), lambda t: (0, 0))],
        out_specs=pl.BlockSpec((_ROW_TILE, hd), lambda t: (t, 0)),
        out_shape=jax.ShapeDtypeStruct((n, hd), jnp.float32),
    )(a, h)


def _topk_out(x, h):
    n, d = x.shape
    hd = h.shape[1]
    return pl.pallas_call(
        _topk_kernel,
        grid=(n // _ROW_TILE,),
        in_specs=[pl.BlockSpec((n, d), lambda t: (0, 0)),
                  pl.BlockSpec((n, hd), lambda t: (0, 0))],
        out_specs=pl.BlockSpec((_ROW_TILE, hd), lambda t: (t, 0)),
        out_shape=jax.ShapeDtypeStruct((n, hd), jnp.float32),
    )(x, h)


def kernel(x, edge_index, edge_type, W1, root1, b1, g1, beta1,
           W2, root2, b2, g2, beta2):
    x = x.astype(jnp.float32)
    ei = edge_index.astype(jnp.int32)
    et = edge_type.astype(jnp.int32)
    e = ei.shape[1]
    keep = jax.random.bernoulli(jax.random.key(123), 0.9, (e,))
    keepf = keep.astype(jnp.float32)
    src, dst = ei[0], ei[1]

    eb = e // _E_TILE
    src2 = src.reshape(eb, _E_TILE)
    dst2 = dst.reshape(eb, _E_TILE)
    et2 = et.reshape(eb, _E_TILE)
    keep2 = keepf.reshape(eb, _E_TILE)

    a = jnp.zeros((_N, _N), jnp.float32).at[src, dst].max(keepf)

    agg1, cnt = _rgcn_agg(src2, dst2, et2, keep2, x)
    h = _rgcn_combine(x, agg1, cnt, W1, root1, b1, g1, beta1)
    agg2, _ = _rgcn_agg(src2, dst2, et2, keep2, h)
    h = _rgcn_combine(h, agg2, cnt, W2, root2, b2, g2, beta2)
    for _ in range(3):
        h = _push(a, h)
    return _topk_out(x, h)


# speedup vs baseline: 11.4083x; 1.0031x over previous
import jax
import jax.numpy as jnp
from jax.experimental import pallas as pl

_N = 2048
_R = 3
_E_TILE = 512
_ROW_TILE = 256


def _agg_kernel(src_ref, dst_ref, et_ref, keep_ref, x_ref, agg_ref, cnt_ref):
    t = pl.program_id(0)

    @pl.when(t == 0)
    def _init():
        agg_ref[...] = jnp.zeros_like(agg_ref)
        cnt_ref[...] = jnp.zeros_like(cnt_ref)

    srow = src_ref[pl.ds(t, 1), :]
    drow = dst_ref[pl.ds(t, 1), :]
    erow = et_ref[pl.ds(t, 1), :]
    krow = keep_ref[pl.ds(t, 1), :]

    ids = jax.lax.broadcasted_iota(jnp.int32, (_N, srow.shape[1]), 0)
    src_oh = ids == srow
    dst_oh = ids == drow
    src_ohT = src_oh.astype(jnp.bfloat16)
    dst_ohT = dst_oh.astype(jnp.bfloat16)
    msg = jax.lax.dot_general(src_ohT, x_ref[...].astype(jnp.bfloat16),
                              (((0,), (0,)), ((), ())),
                              preferred_element_type=jnp.float32)
    msg = msg.astype(jnp.bfloat16)
    for r in range(_R):
        sel = erow == r
        m = jnp.where(sel, krow, 0.0)
        doh = dst_ohT * m.astype(jnp.bfloat16)
        contrib = jnp.dot(doh, msg, preferred_element_type=jnp.float32)
        agg_ref[r] += contrib
        cnt_ref[:, r:r + 1] += jnp.sum(
            dst_oh.astype(jnp.float32) * m, axis=1, keepdims=True)


def _combine_kernel(xin_ref, agg_ref, cnt_ref, w_ref, root_ref, b_ref,
                    g_ref, beta_ref, out_ref):
    z = jnp.dot(xin_ref[...], root_ref[...],
                preferred_element_type=jnp.float32) + b_ref[...]
    for r in range(_R):
        c = jnp.maximum(cnt_ref[:, r:r + 1], 1.0)
        mean_r = agg_ref[r] / c
        z += jnp.dot(mean_r, w_ref[r], preferred_element_type=jnp.float32)
    z = jnp.maximum(z, 0.0)
    mu = jnp.mean(z, axis=1, keepdims=True)
    zc = z - mu
    var = jnp.mean(zc * zc, axis=1, keepdims=True)
    out_ref[...] = zc / jnp.sqrt(var + 1e-5) * g_ref[...] + beta_ref[...]


def _push_kernel(a_ref, h_ref, out_ref):
    t = pl.program_id(0)
    at = a_ref[...]
    h = h_ref[...]
    ht = h_ref[pl.ds(t * _ROW_TILE, _ROW_TILE), :]
    d2 = jnp.sum(at, axis=1, keepdims=True)
    d2 = jnp.where(d2 == 0.0, 1.0, d2)
    prop = jnp.dot(at, h, preferred_element_type=jnp.float32) / d2
    out_ref[...] = 0.9 * ht + 0.1 * prop


def _topk_kernel(x_ref, h_ref, out_ref):
    t = pl.program_id(0)
    k = _N // 2
    xt = x_ref[pl.ds(t * _ROW_TILE, _ROW_TILE), :]
    adj = jax.lax.dot_general(xt, x_ref[...], (((1,), (1,)), ((), ())),
                              preferred_element_type=jnp.float32)
    col = jax.lax.broadcasted_iota(jnp.int32, (_ROW_TILE, _N), 1)
    row = jax.lax.broadcasted_iota(jnp.int32, (_ROW_TILE, _N), 0) + t * _ROW_TILE
    adj = jnp.where(row == col, 0.0, adj)
    deg = jnp.sum(adj, axis=1, keepdims=True)
    deg = jnp.where(deg == 0.0, 1.0, deg)
    p = adj / deg
    u = jax.lax.bitcast_convert_type(p, jnp.int32)
    key = u ^ ((u >> 31) & jnp.int32(0x7FFFFFFF))
    nonneg = key >= 0
    sign_cnt = jnp.sum(nonneg.astype(jnp.int32), axis=1, keepdims=True)
    use_neg = sign_cnt < k
    kk = jnp.where(use_neg, k - sign_cnt, k)
    active = jnp.logical_xor(nonneg, use_neg)
    low = key & jnp.int32(0x7FFFFFFF)
    thr = jnp.zeros((_ROW_TILE, 1), jnp.int32)
    for b in range(30, -1, -1):
        cand = thr | jnp.int32(1 << b)
        ge = jnp.logical_and(active, low >= cand)
        c = jnp.sum(ge.astype(jnp.int32), axis=1, keepdims=True)
        thr = jnp.where(c >= kk, cand, thr)
    tkey = jnp.where(use_neg, thr | jnp.int32(-2147483648), thr)
    sp = jnp.where(key >= tkey, p, 0.0)
    out_ref[...] = jnp.dot(sp, h_ref[...], preferred_element_type=jnp.float32)


def _rgcn_agg(src2, dst2, et2, keep2, xin):
    eb = src2.shape[0]
    n, d = xin.shape
    idx_spec = pl.BlockSpec(src2.shape, lambda t: (0, 0))
    return pl.pallas_call(
        _agg_kernel,
        grid=(eb,),
        in_specs=[idx_spec, idx_spec, idx_spec, idx_spec,
                  pl.BlockSpec((n, d), lambda t: (0, 0))],
        out_specs=[pl.BlockSpec((_R, n, d), lambda t: (0, 0, 0)),
                   pl.BlockSpec((n, 8), lambda t: (0, 0))],
        out_shape=[jax.ShapeDtypeStruct((_R, n, d), jnp.float32),
                   jax.ShapeDtypeStruct((n, 8), jnp.float32)],
    )(src2, dst2, et2, keep2, xin)


def _rgcn_combine(xin, agg, cnt, w, root, b, g, beta):
    n, d = xin.shape
    h = w.shape[2]
    vec = pl.BlockSpec((1, h), lambda t: (0, 0))
    return pl.pallas_call(
        _combine_kernel,
        grid=(n // _ROW_TILE,),
        in_specs=[pl.BlockSpec((_ROW_TILE, d), lambda t: (t, 0)),
                  pl.BlockSpec((_R, _ROW_TILE, d), lambda t: (0, t, 0)),
                  pl.BlockSpec((_ROW_TILE, 8), lambda t: (t, 0)),
                  pl.BlockSpec((_R, d, h), lambda t: (0, 0, 0)),
                  pl.BlockSpec((d, h), lambda t: (0, 0)),
                  vec, vec, vec],
        out_specs=pl.BlockSpec((_ROW_TILE, h), lambda t: (t, 0)),
        out_shape=jax.ShapeDtypeStruct((n, h), jnp.float32),
    )(xin, agg, cnt, w, root, b.reshape(1, h), g.reshape(1, h),
      beta.reshape(1, h))


def _push(a, h):
    n, hd = h.shape
    return pl.pallas_call(
        _push_kernel,
        grid=(n // _ROW_TILE,),
        in_specs=[pl.BlockSpec((_ROW_TILE, n), lambda t: (t, 0)),
                  pl.BlockSpec((n, hd), lambda t: (0, 0))],
        out_specs=pl.BlockSpec((_ROW_TILE, hd), lambda t: (t, 0)),
        out_shape=jax.ShapeDtypeStruct((n, hd), jnp.float32),
    )(a, h)


def _topk_out(x, h):
    n, d = x.shape
    hd = h.shape[1]
    return pl.pallas_call(
        _topk_kernel,
        grid=(n // _ROW_TILE,),
        in_specs=[pl.BlockSpec((n, d), lambda t: (0, 0)),
                  pl.BlockSpec((n, hd), lambda t: (0, 0))],
        out_specs=pl.BlockSpec((_ROW_TILE, hd), lambda t: (t, 0)),
        out_shape=jax.ShapeDtypeStruct((n, hd), jnp.float32),
    )(x, h)


def kernel(x, edge_index, edge_type, W1, root1, b1, g1, beta1,
           W2, root2, b2, g2, beta2):
    x = x.astype(jnp.float32)
    ei = edge_index.astype(jnp.int32)
    et = edge_type.astype(jnp.int32)
    e = ei.shape[1]
    keep = jax.random.bernoulli(jax.random.key(123), 0.9, (e,))
    keepf = keep.astype(jnp.float32)
    src, dst = ei[0], ei[1]

    eb = e // _E_TILE
    src2 = src.reshape(eb, _E_TILE)
    dst2 = dst.reshape(eb, _E_TILE)
    et2 = et.reshape(eb, _E_TILE)
    keep2 = keepf.reshape(eb, _E_TILE)

    a = jnp.zeros((_N, _N), jnp.float32).at[src, dst].max(keepf)

    agg1, cnt = _rgcn_agg(src2, dst2, et2, keep2, x)
    h = _rgcn_combine(x, agg1, cnt, W1, root1, b1, g1, beta1)
    agg2, _ = _rgcn_agg(src2, dst2, et2, keep2, h)
    h = _rgcn_combine(h, agg2, cnt, W2, root2, b2, g2, beta2)
    for _ in range(3):
        h = _push(a, h)
    return _topk_out(x, h)
